# baseline (speedup 1.0000x reference)
import jax
import jax.numpy as jnp
from jax.experimental import pallas as pl
from jax.experimental.pallas import tpu as pltpu


def _qk_body(x_ref, w_ref, b_ref, out_ref):
    out_ref[...] = (
        jnp.dot(x_ref[...], w_ref[...], preferred_element_type=jnp.float32)
        + b_ref[...]
    )


def _qk_matmul(x, W, b):
    n, d = x.shape
    m = W.shape[1]
    blk = 2000
    return pl.pallas_call(
        _qk_body,
        out_shape=jax.ShapeDtypeStruct((n, m), jnp.float32),
        grid=(n // blk,),
        in_specs=[
            pl.BlockSpec((blk, d), lambda i: (i, 0)),
            pl.BlockSpec((d, m), lambda i: (0, 0)),
            pl.BlockSpec((1, m), lambda i: (0, 0)),
        ],
        out_specs=pl.BlockSpec((blk, m), lambda i: (i, 0)),
    )(x, W, b.reshape(1, m))


def kernel(x, edge_index, Wq, bq, Wk, bk):
    W = jnp.concatenate([Wq, Wk], axis=1)
    b = jnp.concatenate([bq, bk])
    qk = _qk_matmul(x, W, b)
    q = qk[:, : Wq.shape[1]]
    k = qk[:, Wq.shape[1] :]
    src = edge_index[0]
    dst = edge_index[1]
    scores = jnp.sum(q[src] * k[dst], axis=1)
    seg_max = jax.ops.segment_max(scores, src, num_segments=x.shape[0])
    seg_max = jnp.where(jnp.isfinite(seg_max), seg_max, 0.0)
    ex = jnp.exp(scores - seg_max[src])
    denom = jax.ops.segment_sum(ex, src, num_segments=x.shape[0])
    attn = ex / denom[src]
    return (x, attn)


# speedup vs baseline: 6.2225x; 6.2225x over previous
import functools

import jax
import jax.numpy as jnp
from jax import lax
from jax.experimental import pallas as pl
from jax.experimental.pallas import tpu as pltpu
from jax.experimental.pallas import tpu_sc as plsc

NC = 2
NS = 16
NW = NC * NS
LANES = 16

N_NODES = 10000
N_EDGES = 160000
D = 128

E_W = N_EDGES // NW
CHUNK2 = 128
LAST2 = E_W - CHUNK2
N_CH2 = E_W // CHUNK2 + 1
CHUNK4 = 40
N_CH4 = E_W // CHUNK4

_mesh = plsc.VectorSubcoreMesh(core_axis_name="c", subcore_axis_name="s")


def _qk_body(x_ref, wq_ref, bq_ref, wk_ref, bk_ref, q_ref, k_ref):
    xb = x_ref[...]
    q_ref[...] = jnp.dot(xb, wq_ref[...], preferred_element_type=jnp.float32) + bq_ref[...]
    k_ref[...] = jnp.dot(xb, wk_ref[...], preferred_element_type=jnp.float32) + bk_ref[...]


def _qk_matmul(x, Wq, bq, Wk, bk):
    n, dm = x.shape
    m = Wq.shape[1]
    blk = 2000
    return pl.pallas_call(
        _qk_body,
        out_shape=(
            jax.ShapeDtypeStruct((n, m), jnp.float32),
            jax.ShapeDtypeStruct((n, m), jnp.float32),
        ),
        grid=(n // blk,),
        in_specs=[
            pl.BlockSpec((blk, dm), lambda i: (i, 0)),
            pl.BlockSpec((dm, m), lambda i: (0, 0)),
            pl.BlockSpec((1, m), lambda i: (0, 0)),
            pl.BlockSpec((dm, m), lambda i: (0, 0)),
            pl.BlockSpec((1, m), lambda i: (0, 0)),
        ],
        out_specs=(
            pl.BlockSpec((blk, m), lambda i: (i, 0)),
            pl.BlockSpec((blk, m), lambda i: (i, 0)),
        ),
    )(x, Wq, bq.reshape(1, m), Wk, bk.reshape(1, m))


@functools.partial(
    pl.kernel,
    mesh=_mesh,
    out_type=jax.ShapeDtypeStruct((N_EDGES, LANES), jnp.float32),
    scratch_types=[
        pltpu.VMEM((CHUNK2,), jnp.int32),
        pltpu.VMEM((CHUNK2,), jnp.int32),
        pltpu.VMEM((CHUNK2, D), jnp.float32),
        pltpu.VMEM((CHUNK2, D), jnp.float32),
        pltpu.VMEM((CHUNK2, LANES), jnp.float32),
        pltpu.SemaphoreType.DMA,
        pltpu.SemaphoreType.DMA,
    ],
)
def _sc_partials(q_hbm, k_hbm, src_hbm, dst_hbm, p_hbm,
                 srcb, dstb, qb, kb, pb, sem1, sem2):
    wid = lax.axis_index("s") * NC + lax.axis_index("c")
    base_w = wid * E_W

    def chunk_body(ci, _):
        off = jnp.minimum(ci * CHUNK2, LAST2)
        base = base_w + off
        pltpu.sync_copy(src_hbm.at[pl.ds(base, CHUNK2)], srcb)
        pltpu.sync_copy(dst_hbm.at[pl.ds(base, CHUNK2)], dstb)
        cq = pltpu.async_copy(q_hbm.at[srcb], qb, sem1)
        ck = pltpu.async_copy(k_hbm.at[dstb], kb, sem2)
        cq.wait()
        ck.wait()

        def group_body(gi, _):
            goff = gi * LANES
            for el in range(LANES):
                e = goff + el
                p = qb[e, pl.ds(0, LANES)] * kb[e, pl.ds(0, LANES)]
                for j in range(1, D // LANES):
                    p = p + qb[e, pl.ds(j * LANES, LANES)] * kb[e, pl.ds(j * LANES, LANES)]
                pb[e, pl.ds(0, LANES)] = p
            return 0

        lax.fori_loop(0, CHUNK2 // LANES, group_body, 0, unroll=False)
        pltpu.sync_copy(pb, p_hbm.at[pl.ds(base, CHUNK2)])
        return 0

    lax.fori_loop(0, N_CH2, chunk_body, 0, unroll=False)


def _exp_body(p_ref, ex_ref):
    gi = lax.broadcasted_iota(jnp.int32, (2048, 128), 0)
    gc = lax.broadcasted_iota(jnp.int32, (2048, 128), 1)
    G = jnp.where(gi // LANES == gc, 1.0, 0.0).astype(jnp.float32)
    s = jnp.dot(p_ref[...], G, preferred_element_type=jnp.float32,
                precision=jax.lax.Precision.HIGHEST)
    m = 0.5 * (jnp.max(s) + jnp.min(s))
    ex_ref[...] = jnp.exp(s - m)


def _tc_exp(p2):
    return pl.pallas_call(
        _exp_body,
        out_shape=jax.ShapeDtypeStruct((N_EDGES // 128, 128), jnp.float32),
    )(p2)


@functools.partial(
    pl.kernel,
    mesh=_mesh,
    out_type=(
        jax.ShapeDtypeStruct((N_NODES,), jnp.float32),
        jax.ShapeDtypeStruct((N_NODES,), jnp.float32),
    ),
    scratch_types=[
        pltpu.VMEM((N_CH4, CHUNK4), jnp.float32),
        pltpu.VMEM((N_CH4, CHUNK4), jnp.int32),
        pltpu.VMEM_SHARED((N_NODES,), jnp.float32),
    ],
)
def _sc_denoms(ex_hbm, src_hbm, zeros_hbm, d0_hbm, d1_hbm,
               exb, srcb, denom_sh):
    cid = lax.axis_index("c")
    sid = lax.axis_index("s")
    wid = sid * NC + cid

    @pl.when(sid == 0)
    def _():
        pltpu.sync_copy(zeros_hbm, denom_sh)

    plsc.subcore_barrier()

    pltpu.sync_copy(ex_hbm.at[wid], exb)
    pltpu.sync_copy(src_hbm.at[wid], srcb)

    def chunk_body(ci, _):
        pltpu.sync_copy(exb.at[ci], denom_sh.at[srcb.at[ci]], add=True)
        return 0

    lax.fori_loop(0, N_CH4, chunk_body, 0, unroll=False)

    plsc.subcore_barrier()

    @pl.when((sid == 0) & (cid == 0))
    def _():
        pltpu.sync_copy(denom_sh, d0_hbm)

    @pl.when((sid == 0) & (cid == 1))
    def _():
        pltpu.sync_copy(denom_sh, d1_hbm)


@functools.partial(
    pl.kernel,
    mesh=_mesh,
    out_type=jax.ShapeDtypeStruct((N_EDGES,), jnp.float32),
    scratch_types=[
        pltpu.VMEM((CHUNK2,), jnp.float32),
        pltpu.VMEM((CHUNK2,), jnp.int32),
        pltpu.VMEM((CHUNK2,), jnp.float32),
        pltpu.VMEM((CHUNK2,), jnp.float32),
        pltpu.VMEM((CHUNK2,), jnp.float32),
        pltpu.SemaphoreType.DMA,
        pltpu.SemaphoreType.DMA,
    ],
)
def _sc_normalize(ex_hbm, src_hbm, d0_hbm, d1_hbm, attn_hbm,
                  exb, srcb, g0, g1, ab, sem1, sem2):
    wid = lax.axis_index("s") * NC + lax.axis_index("c")
    base_w = wid * E_W

    def chunk_body(ci, _):
        base = base_w + jnp.minimum(ci * CHUNK2, LAST2)
        pltpu.sync_copy(ex_hbm.at[pl.ds(base, CHUNK2)], exb)
        pltpu.sync_copy(src_hbm.at[pl.ds(base, CHUNK2)], srcb)
        c0 = pltpu.async_copy(d0_hbm.at[srcb], g0, sem1)
        c1 = pltpu.async_copy(d1_hbm.at[srcb], g1, sem2)
        c0.wait()
        c1.wait()

        def group_body(gi, _):
            sl = pl.ds(gi * LANES, LANES)
            ab[sl] = exb[sl] / (g0[sl] + g1[sl])
            return 0

        lax.fori_loop(0, CHUNK2 // LANES, group_body, 0, unroll=False)
        pltpu.sync_copy(ab, attn_hbm.at[pl.ds(base, CHUNK2)])
        return 0

    lax.fori_loop(0, N_CH2, chunk_body, 0, unroll=False)


def kernel(x, edge_index, Wq, bq, Wk, bk):
    src = edge_index[0]
    dst = edge_index[1]
    q, k = _qk_matmul(x, Wq, bq, Wk, bk)
    p = _sc_partials(q, k, src, dst)
    ex = _tc_exp(p.reshape(N_EDGES // 128, 128 * LANES)).reshape(N_EDGES)
    zeros = jnp.zeros((N_NODES,), jnp.float32)
    d0, d1 = _sc_denoms(
        ex.reshape(NW, N_CH4, CHUNK4), src.reshape(NW, N_CH4, CHUNK4), zeros)
    attn = _sc_normalize(ex, src, d0, d1)
    return (x, attn)


# speedup vs baseline: 13.3065x; 2.1385x over previous
import functools

import jax
import jax.numpy as jnp
from jax import lax
from jax.experimental import pallas as pl
from jax.experimental.pallas import tpu as pltpu
from jax.experimental.pallas import tpu_sc as plsc

NC = 2
NS = 16
NW = NC * NS
LANES = 16

N_NODES = 10000
N_EDGES = 160000
D = 128

E_W = N_EDGES // NW
CHUNK2 = 128
LAST2 = E_W - CHUNK2
N_CH2 = E_W // CHUNK2 + 1
CHUNK4 = 40
N_CH4 = E_W // CHUNK4

_mesh = plsc.VectorSubcoreMesh(core_axis_name="c", subcore_axis_name="s")


def _qk_body(x_ref, wq_ref, bq_ref, wk_ref, bk_ref, q_ref, k_ref):
    xb = x_ref[...]
    q_ref[...] = jnp.dot(xb, wq_ref[...], preferred_element_type=jnp.float32) + bq_ref[...]
    k_ref[...] = jnp.dot(xb, wk_ref[...], preferred_element_type=jnp.float32) + bk_ref[...]


def _qk_matmul(x, Wq, bq, Wk, bk):
    n, dm = x.shape
    m = Wq.shape[1]
    blk = 2000
    return pl.pallas_call(
        _qk_body,
        out_shape=(
            jax.ShapeDtypeStruct((n, m), jnp.float32),
            jax.ShapeDtypeStruct((n, m), jnp.float32),
        ),
        grid=(n // blk,),
        in_specs=[
            pl.BlockSpec((blk, dm), lambda i: (i, 0)),
            pl.BlockSpec((dm, m), lambda i: (0, 0)),
            pl.BlockSpec((1, m), lambda i: (0, 0)),
            pl.BlockSpec((dm, m), lambda i: (0, 0)),
            pl.BlockSpec((1, m), lambda i: (0, 0)),
        ],
        out_specs=(
            pl.BlockSpec((blk, m), lambda i: (i, 0)),
            pl.BlockSpec((blk, m), lambda i: (i, 0)),
        ),
    )(x, Wq, bq.reshape(1, m), Wk, bk.reshape(1, m))


@functools.partial(
    pl.kernel,
    mesh=_mesh,
    out_type=jax.ShapeDtypeStruct((N_EDGES * LANES,), jnp.float32),
    scratch_types=[
        pltpu.VMEM((E_W,), jnp.int32),
        pltpu.VMEM((E_W,), jnp.int32),
        pltpu.VMEM((CHUNK2, D), jnp.float32),
        pltpu.VMEM((CHUNK2, D), jnp.float32),
        pltpu.VMEM((CHUNK2, D), jnp.float32),
        pltpu.VMEM((CHUNK2, D), jnp.float32),
        pltpu.VMEM((CHUNK2 * LANES,), jnp.float32),
        pltpu.VMEM((CHUNK2 * LANES,), jnp.float32),
        pltpu.SemaphoreType.DMA,
        pltpu.SemaphoreType.DMA,
        pltpu.SemaphoreType.DMA,
        pltpu.SemaphoreType.DMA,
        pltpu.SemaphoreType.DMA,
        pltpu.SemaphoreType.DMA,
    ],
)
def _sc_partials(q_hbm, k_hbm, src2_hbm, dst2_hbm, p_hbm,
                 srcall, dstall, qbA, kbA, qbB, kbB, pbA, pbB,
                 sqA, skA, sqB, skB, spA, spB):
    wid = lax.axis_index("s") * NC + lax.axis_index("c")
    base_w = wid * E_W
    pltpu.sync_copy(src2_hbm.at[wid], srcall)
    pltpu.sync_copy(dst2_hbm.at[wid], dstall)

    def _off(ci):
        return jnp.minimum(ci * CHUNK2, LAST2)

    def _start(ci, qb, kb, sq, sk):
        o = _off(ci)
        pltpu.async_copy(q_hbm.at[srcall.at[pl.ds(o, CHUNK2)]], qb, sq)
        pltpu.async_copy(k_hbm.at[dstall.at[pl.ds(o, CHUNK2)]], kb, sk)

    def _wait_rows(qb, kb, sq, sk):
        pltpu.make_async_copy(
            q_hbm.at[srcall.at[pl.ds(0, CHUNK2)]], qb, sq).wait()
        pltpu.make_async_copy(
            k_hbm.at[dstall.at[pl.ds(0, CHUNK2)]], kb, sk).wait()

    def _compute(qb, kb, pb):
        def group_body(gi, _):
            goff = gi * LANES
            for el in range(LANES):
                e = goff + el
                p = qb[e, pl.ds(0, LANES)] * kb[e, pl.ds(0, LANES)]
                for j in range(1, D // LANES):
                    p = p + qb[e, pl.ds(j * LANES, LANES)] * kb[e, pl.ds(j * LANES, LANES)]
                pb[pl.ds(e * LANES, LANES)] = p
            return 0

        lax.fori_loop(0, CHUNK2 // LANES, group_body, 0, unroll=False)

    def _wout(ci, pb, sp):
        b16 = (base_w + _off(ci)) * LANES
        pltpu.async_copy(pb, p_hbm.at[pl.ds(b16, CHUNK2 * LANES)], sp)

    def _wout_drain(pb, sp):
        pltpu.make_async_copy(
            pb, p_hbm.at[pl.ds(0, CHUNK2 * LANES)], sp).wait()

    _start(0, qbA, kbA, sqA, skA)

    def pair_body(cj, _):
        c0 = 2 * cj
        _start(c0 + 1, qbB, kbB, sqB, skB)
        _wait_rows(qbA, kbA, sqA, skA)

        @pl.when(cj > 0)
        def _():
            _wout_drain(pbA, spA)

        _compute(qbA, kbA, pbA)
        _wout(c0, pbA, spA)
        _start(c0 + 2, qbA, kbA, sqA, skA)
        _wait_rows(qbB, kbB, sqB, skB)

        @pl.when(cj > 0)
        def _():
            _wout_drain(pbB, spB)

        _compute(qbB, kbB, pbB)
        _wout(c0 + 1, pbB, spB)
        return 0

    lax.fori_loop(0, N_CH2 // 2, pair_body, 0, unroll=False)
    _wait_rows(qbA, kbA, sqA, skA)
    _wout_drain(pbA, spA)
    _wout_drain(pbB, spB)


def _exp_body(p_ref, ex_ref):
    gi = lax.broadcasted_iota(jnp.int32, (2048, 128), 0)
    gc = lax.broadcasted_iota(jnp.int32, (2048, 128), 1)
    G = jnp.where(gi // LANES == gc, 1.0, 0.0).astype(jnp.float32)
    s = jnp.dot(p_ref[...], G, preferred_element_type=jnp.float32,
                precision=jax.lax.Precision.HIGHEST)
    m = 0.5 * (jnp.max(s) + jnp.min(s))
    ex_ref[...] = jnp.exp(s - m)


def _tc_exp(p2):
    return pl.pallas_call(
        _exp_body,
        out_shape=jax.ShapeDtypeStruct((N_EDGES // 128, 128), jnp.float32),
    )(p2)


@functools.partial(
    pl.kernel,
    mesh=_mesh,
    out_type=(
        jax.ShapeDtypeStruct((N_NODES,), jnp.float32),
        jax.ShapeDtypeStruct((N_NODES,), jnp.float32),
    ),
    scratch_types=[
        pltpu.VMEM((N_CH4, CHUNK4), jnp.float32),
        pltpu.VMEM((N_CH4, CHUNK4), jnp.int32),
        pltpu.VMEM_SHARED((N_NODES,), jnp.float32),
    ],
)
def _sc_denoms(ex_hbm, src_hbm, zeros_hbm, d0_hbm, d1_hbm,
               exb, srcb, denom_sh):
    cid = lax.axis_index("c")
    sid = lax.axis_index("s")
    wid = sid * NC + cid

    @pl.when(sid == 0)
    def _():
        pltpu.sync_copy(zeros_hbm, denom_sh)

    plsc.subcore_barrier()

    pltpu.sync_copy(ex_hbm.at[wid], exb)
    pltpu.sync_copy(src_hbm.at[wid], srcb)

    def chunk_body(ci, _):
        pltpu.sync_copy(exb.at[ci], denom_sh.at[srcb.at[ci]], add=True)
        return 0

    lax.fori_loop(0, N_CH4, chunk_body, 0, unroll=False)

    plsc.subcore_barrier()

    @pl.when((sid == 0) & (cid == 0))
    def _():
        pltpu.sync_copy(denom_sh, d0_hbm)

    @pl.when((sid == 0) & (cid == 1))
    def _():
        pltpu.sync_copy(denom_sh, d1_hbm)


@functools.partial(
    pl.kernel,
    mesh=_mesh,
    out_type=jax.ShapeDtypeStruct((N_EDGES,), jnp.float32),
    scratch_types=[
        pltpu.VMEM((E_W,), jnp.float32),
        pltpu.VMEM((E_W,), jnp.int32),
        pltpu.VMEM((E_W,), jnp.float32),
        pltpu.VMEM((E_W,), jnp.float32),
        pltpu.VMEM((E_W,), jnp.float32),
        pltpu.SemaphoreType.DMA,
        pltpu.SemaphoreType.DMA,
    ],
)
def _sc_normalize(ex2_hbm, src2_hbm, d0_hbm, d1_hbm, attn_hbm,
                  exall, srcall, g0, g1, ab, s0, s1):
    wid = lax.axis_index("s") * NC + lax.axis_index("c")
    base_w = wid * E_W
    pltpu.sync_copy(ex2_hbm.at[wid], exall)
    pltpu.sync_copy(src2_hbm.at[wid], srcall)
    for ci in range(N_CH2):
        o = min(ci * CHUNK2, LAST2)
        idx = srcall.at[pl.ds(o, CHUNK2)]
        pltpu.async_copy(d0_hbm.at[idx], g0.at[pl.ds(o, CHUNK2)], s0)
        pltpu.async_copy(d1_hbm.at[idx], g1.at[pl.ds(o, CHUNK2)], s1)
    for ci in range(N_CH2):
        idx = srcall.at[pl.ds(0, CHUNK2)]
        pltpu.make_async_copy(
            d0_hbm.at[idx], g0.at[pl.ds(0, CHUNK2)], s0).wait()
        pltpu.make_async_copy(
            d1_hbm.at[idx], g1.at[pl.ds(0, CHUNK2)], s1).wait()

    def group_body(gi, _):
        sl = pl.ds(jnp.minimum(gi * LANES, E_W - LANES), LANES)
        ab[sl] = exall[sl] / (g0[sl] + g1[sl])
        return 0

    lax.fori_loop(0, E_W // LANES + 1, group_body, 0, unroll=False)
    pltpu.sync_copy(ab, attn_hbm.at[pl.ds(base_w, E_W)])


def kernel(x, edge_index, Wq, bq, Wk, bk):
    src = edge_index[0]
    dst = edge_index[1]
    src2 = src.reshape(NW, E_W)
    dst2 = dst.reshape(NW, E_W)
    q, k = _qk_matmul(x, Wq, bq, Wk, bk)
    p = _sc_partials(q, k, src2, dst2)
    ex = _tc_exp(p.reshape(N_EDGES // 128, 128 * LANES)).reshape(N_EDGES)
    zeros = jnp.zeros((N_NODES,), jnp.float32)
    d0, d1 = _sc_denoms(
        ex.reshape(NW, N_CH4, CHUNK4), src.reshape(NW, N_CH4, CHUNK4), zeros)
    attn = _sc_normalize(ex.reshape(NW, E_W), src2, d0, d1)
    return (x, attn)
